# baseline (speedup 1.0000x reference)
import functools

import jax
import jax.numpy as jnp
from jax import lax
from jax.experimental import pallas as pl
from jax.experimental.pallas import tpu as pltpu
from jax.experimental.pallas import tpu_sc as plsc

NC = 2
NS = 16
L = 16
K = 80

_mesh = functools.partial(
    plsc.VectorSubcoreMesh, core_axis_name="c", subcore_axis_name="s",
    num_cores=NC, num_subcores=NS)


def _fill(ref, n, value):
  v = jnp.full((L,), value, ref.dtype)
  for k in range(n // L):
    ref[pl.ds(k * L, L)] = v


def _make_deg_kernel(e):
  npad = 10240
  per_w = e // (NC * NS)
  chunks = per_w // K
  assert per_w % K == 0

  @functools.partial(
      pl.kernel,
      out_type=jax.ShapeDtypeStruct((NC * 2 * npad,), jnp.float32),
      mesh=_mesh(),
      scratch_types=[
          pltpu.VMEM((K,), jnp.int32),
          pltpu.VMEM((K,), jnp.int32),
          pltpu.VMEM((K,), jnp.float32),
          pltpu.VMEM((640,), jnp.float32),
          pltpu.VMEM_SHARED((npad,), jnp.float32),
          pltpu.VMEM_SHARED((npad,), jnp.float32),
      ],
  )
  def deg_kernel(src_ids, dst_ids, out, sidx, didx, ones, cbuf, h_out, h_in):
    c = lax.axis_index("c")
    s = lax.axis_index("s")
    _fill(ones, K, 1.0)
    _fill(cbuf, 640, 0.0)
    pltpu.sync_copy(cbuf, h_out.at[pl.ds(s * 640, 640)])
    pltpu.sync_copy(cbuf, h_in.at[pl.ds(s * 640, 640)])
    plsc.subcore_barrier()

    base0 = (c * NS + s) * per_w

    def body(i, carry):
      base = base0 + i * K
      pltpu.sync_copy(src_ids.at[pl.ds(base, K)], sidx)
      pltpu.sync_copy(dst_ids.at[pl.ds(base, K)], didx)
      pltpu.sync_copy(ones, h_out.at[sidx], add=True)
      pltpu.sync_copy(ones, h_in.at[didx], add=True)
      return carry

    lax.fori_loop(0, chunks, body, 0)
    plsc.subcore_barrier()
    pltpu.sync_copy(h_out.at[pl.ds(s * 640, 640)], cbuf)
    pltpu.sync_copy(cbuf, out.at[pl.ds(c * 2 * npad + s * 640, 640)])
    pltpu.sync_copy(h_in.at[pl.ds(s * 640, 640)], cbuf)
    pltpu.sync_copy(cbuf, out.at[pl.ds(c * 2 * npad + npad + s * 640, 640)])

  return deg_kernel


def _make_agg_kernel(n, e, hc, edge_split=False):
  KC = 64
  NB = 4
  per_t = e // (NC * NS) if edge_split else e // NS
  g = per_t // (NB * KC)
  assert per_t % (NB * KC) == 0
  npad = 10240
  assert npad >= n
  rows_t = npad // NS
  cp = 64
  assert rows_t % cp == 0

  @functools.partial(
      pl.kernel,
      out_type=jax.ShapeDtypeStruct((NC, npad, hc), jnp.float32),
      mesh=_mesh(),
      scratch_types=[
          [pltpu.VMEM((KC,), jnp.int32) for _ in range(NB)],
          [pltpu.VMEM((KC,), jnp.int32) for _ in range(NB)],
          [pltpu.VMEM((KC, hc), jnp.float32) for _ in range(NB)],
          pltpu.VMEM((cp, hc), jnp.float32),
          pltpu.VMEM_SHARED((npad, hc), jnp.float32),
          [pltpu.SemaphoreType.DMA for _ in range(NB)],
          [pltpu.SemaphoreType.DMA for _ in range(NB)],
      ],
  )
  def agg_kernel(h, src_ids, dst_ids, out, sidx, didx, rows, cbuf, acc,
                 gsem, ssem):
    c = lax.axis_index("c")
    s = lax.axis_index("s")
    hsrc = h if edge_split else h.at[c]

    def zbody(r, carry):
      z = jnp.zeros((L,), jnp.float32)
      for k in range(hc // L):
        cbuf[r, pl.ds(k * L, L)] = z
      return carry
    lax.fori_loop(0, cp, zbody, 0)
    for j in range(rows_t // cp):
      pltpu.sync_copy(cbuf, acc.at[pl.ds(s * rows_t + j * cp, cp)])
    plsc.subcore_barrier()

    base0 = ((c * NS + s) if edge_split else s) * per_t

    def body(i, carry):
      base = base0 + i * (NB * KC)
      gathers = []
      for j in range(NB):
        pltpu.sync_copy(src_ids.at[pl.ds(base + j * KC, KC)], sidx[j])
        gathers.append(pltpu.async_copy(hsrc.at[sidx[j]], rows[j], gsem[j]))
      for j in range(NB):
        pltpu.sync_copy(dst_ids.at[pl.ds(base + j * KC, KC)], didx[j])
      scatters = []
      for j in range(NB):
        gathers[j].wait()
        scatters.append(
            pltpu.async_copy(rows[j], acc.at[didx[j]], ssem[j], add=True))
      for sc in scatters:
        sc.wait()
      return carry

    lax.fori_loop(0, g, body, 0)
    plsc.subcore_barrier()
    for j in range(rows_t // cp):
      r0 = s * rows_t + j * cp
      pltpu.sync_copy(acc.at[pl.ds(r0, cp)], cbuf)
      pltpu.sync_copy(cbuf, out.at[c].at[pl.ds(r0, cp)])

  return agg_kernel


def _prep_body(degp_ref, x_ref, norms_ref, s0_ref):
  n = x_ref.shape[0]
  d = degp_ref[:, 0:2] + degp_ref[:, 2:4]
  norm = jnp.where(d > 0.0, lax.rsqrt(jnp.maximum(d, 1e-30)), 0.0)
  norms_ref[...] = norm
  s0_ref[...] = x_ref[...] * norm[:n, 0:1]


def _layer_body(agg_ref, norms_ref, w_ref, b_ref, g_ref, bt_ref, out_ref,
                *, concat):
  n = norms_ref.shape[0]
  if concat:
    a = jnp.concatenate([agg_ref[0], agg_ref[1]], axis=1)[:n]
  else:
    a = (agg_ref[0] + agg_ref[1])[:n]
  a = a * norms_ref[:, 1:2]
  z = jnp.dot(a, w_ref[...], preferred_element_type=jnp.float32) + b_ref[...]
  mu = jnp.mean(z, axis=0, keepdims=True)
  xc = z - mu
  var = jnp.mean(xc * xc, axis=0, keepdims=True)
  h = g_ref[...] * xc * lax.rsqrt(var + 1e-5) + bt_ref[...]
  h = jnp.maximum(h, 0.0)
  s = h * norms_ref[:, 0:1]
  half = s.shape[1] // 2
  out_ref[0] = s[:, :half]
  out_ref[1] = s[:, half:]


def _final_body(agg_ref, norms_ref, w_ref, b_ref, g_ref, bt_ref,
                wr0_ref, br0_ref, wr1_ref, br1_ref, wr2_ref, br2_ref,
                out_ref):
  n = norms_ref.shape[0]
  a = jnp.concatenate([agg_ref[0], agg_ref[1]], axis=1)[:n]
  a = a * norms_ref[:, 1:2]
  z = jnp.dot(a, w_ref[...], preferred_element_type=jnp.float32) + b_ref[...]
  mu = jnp.mean(z, axis=0, keepdims=True)
  xc = z - mu
  var = jnp.mean(xc * xc, axis=0, keepdims=True)
  h = g_ref[...] * xc * lax.rsqrt(var + 1e-5) + bt_ref[...]
  h = jnp.maximum(h, 0.0)
  hg = jnp.mean(h, axis=0, keepdims=True)
  y = jnp.maximum(jnp.dot(hg, wr0_ref[...],
                          preferred_element_type=jnp.float32) + br0_ref[...], 0.0)
  y = jnp.maximum(jnp.dot(y, wr1_ref[...],
                          preferred_element_type=jnp.float32) + br1_ref[...], 0.0)
  y = jnp.dot(y, wr2_ref[...], preferred_element_type=jnp.float32) + br2_ref[...]
  m = jnp.max(y, axis=0, keepdims=True)
  lse = m + jnp.log(jnp.sum(jnp.exp(y - m), axis=0, keepdims=True))
  out_ref[...] = y - lse


def kernel(x, edge_index, W0, b0, W1, b1, W2, b2, W3, b3, gamma, beta,
           Wr0, br0, Wr1, br1, Wr2, br2):
  n, in_feats = x.shape
  e = edge_index.shape[1]
  h_dim = W0.shape[1]
  npad = 10240

  src_ids = edge_index[0]
  dst_ids = edge_index[1]
  degp = _make_deg_kernel(e)(src_ids, dst_ids)
  degp_col = jnp.transpose(degp.reshape(4, npad))
  norms_pad, s0 = pl.pallas_call(
      _prep_body,
      out_shape=[
          jax.ShapeDtypeStruct((npad, 2), jnp.float32),
          jax.ShapeDtypeStruct((n, in_feats), jnp.float32),
      ],
  )(degp_col, x)
  norms = norms_pad[:n]

  epad = 327680
  assert epad % (NC * NS * 4 * 64) == 0 and epad >= e
  p = epad - e
  pad_src = (jnp.arange(p, dtype=jnp.int32) * 131) % n
  pad_dst = n + (jnp.arange(p, dtype=jnp.int32) % (npad - n))
  src_p = jnp.concatenate([src_ids, pad_src])
  dst_p = jnp.concatenate([dst_ids, pad_dst])
  agg_in = _make_agg_kernel(n, epad, in_feats, edge_split=True)
  agg_h = _make_agg_kernel(n, epad, h_dim // 2)
  out_split = jax.ShapeDtypeStruct((NC, n, h_dim // 2), jnp.float32)
  layer0_call = pl.pallas_call(
      functools.partial(_layer_body, concat=False), out_shape=out_split)
  layer_call = pl.pallas_call(
      functools.partial(_layer_body, concat=True), out_shape=out_split)

  a0 = agg_in(s0, src_p, dst_p)
  s1 = layer0_call(a0, norms, W0, b0, gamma, beta)
  a1 = agg_h(s1, src_p, dst_p)
  s2 = layer_call(a1, norms, W1, b1, gamma, beta)
  a2 = agg_h(s2, src_p, dst_p)
  s3 = layer_call(a2, norms, W2, b2, gamma, beta)
  a3 = agg_h(s3, src_p, dst_p)

  out = pl.pallas_call(
      _final_body,
      out_shape=jax.ShapeDtypeStruct((1, Wr2.shape[1]), jnp.float32),
  )(a3, norms, W3, b3, gamma, beta, Wr0, br0, Wr1, br1, Wr2, br2)
  return out


# speedup vs baseline: 1.2567x; 1.2567x over previous
import functools

import jax
import jax.numpy as jnp
from jax import lax
from jax.experimental import pallas as pl
from jax.experimental.pallas import tpu as pltpu
from jax.experimental.pallas import tpu_sc as plsc

NC = 2
NS = 16
L = 16
K = 80

_mesh = functools.partial(
    plsc.VectorSubcoreMesh, core_axis_name="c", subcore_axis_name="s",
    num_cores=NC, num_subcores=NS)


def _fill(ref, n, value):
  v = jnp.full((L,), value, ref.dtype)
  for k in range(n // L):
    ref[pl.ds(k * L, L)] = v


def _make_deg_kernel(e):
  npad = 10240
  per_w = e // (NC * NS)
  chunks = per_w // K
  assert per_w % K == 0

  @functools.partial(
      pl.kernel,
      out_type=jax.ShapeDtypeStruct((NC * 2 * npad,), jnp.float32),
      mesh=_mesh(),
      scratch_types=[
          pltpu.VMEM((K,), jnp.int32),
          pltpu.VMEM((K,), jnp.int32),
          pltpu.VMEM((K,), jnp.float32),
          pltpu.VMEM((640,), jnp.float32),
          pltpu.VMEM_SHARED((npad,), jnp.float32),
          pltpu.VMEM_SHARED((npad,), jnp.float32),
      ],
  )
  def deg_kernel(src_ids, dst_ids, out, sidx, didx, ones, cbuf, h_out, h_in):
    c = lax.axis_index("c")
    s = lax.axis_index("s")
    _fill(ones, K, 1.0)
    _fill(cbuf, 640, 0.0)
    pltpu.sync_copy(cbuf, h_out.at[pl.ds(s * 640, 640)])
    pltpu.sync_copy(cbuf, h_in.at[pl.ds(s * 640, 640)])
    plsc.subcore_barrier()

    base0 = (c * NS + s) * per_w

    def body(i, carry):
      base = base0 + i * K
      pltpu.sync_copy(src_ids.at[pl.ds(base, K)], sidx)
      pltpu.sync_copy(dst_ids.at[pl.ds(base, K)], didx)
      pltpu.sync_copy(ones, h_out.at[sidx], add=True)
      pltpu.sync_copy(ones, h_in.at[didx], add=True)
      return carry

    lax.fori_loop(0, chunks, body, 0)
    plsc.subcore_barrier()
    pltpu.sync_copy(h_out.at[pl.ds(s * 640, 640)], cbuf)
    pltpu.sync_copy(cbuf, out.at[pl.ds(c * 2 * npad + s * 640, 640)])
    pltpu.sync_copy(h_in.at[pl.ds(s * 640, 640)], cbuf)
    pltpu.sync_copy(cbuf, out.at[pl.ds(c * 2 * npad + npad + s * 640, 640)])

  return deg_kernel


def _make_agg_kernel(n, e, hc, edge_split=False):
  KC = 128
  NB = 2
  per_t = e // (NC * NS) if edge_split else e // NS
  g = per_t // (NB * KC)
  assert per_t % (NB * KC) == 0
  npad = 10240
  assert npad >= n
  rows_t = npad // NS
  cp = 64
  assert rows_t % cp == 0

  @functools.partial(
      pl.kernel,
      out_type=jax.ShapeDtypeStruct((NC, npad, hc), jnp.float32),
      mesh=_mesh(),
      scratch_types=[
          [pltpu.VMEM((KC,), jnp.int32) for _ in range(NB)],
          [pltpu.VMEM((KC,), jnp.int32) for _ in range(NB)],
          [pltpu.VMEM((KC, hc), jnp.float32) for _ in range(NB)],
          pltpu.VMEM((cp, hc), jnp.float32),
          pltpu.VMEM_SHARED((npad, hc), jnp.float32),
          [pltpu.SemaphoreType.DMA for _ in range(NB)],
          [pltpu.SemaphoreType.DMA for _ in range(NB)],
      ],
  )
  def agg_kernel(h, src_ids, dst_ids, out, sidx, didx, rows, cbuf, acc,
                 gsem, ssem):
    c = lax.axis_index("c")
    s = lax.axis_index("s")
    hsrc = h if edge_split else h.at[c]

    def zbody(r, carry):
      z = jnp.zeros((L,), jnp.float32)
      for k in range(hc // L):
        cbuf[r, pl.ds(k * L, L)] = z
      return carry
    lax.fori_loop(0, cp, zbody, 0)
    for j in range(rows_t // cp):
      pltpu.sync_copy(cbuf, acc.at[pl.ds(s * rows_t + j * cp, cp)])
    plsc.subcore_barrier()

    base0 = ((c * NS + s) if edge_split else s) * per_t

    def body(i, carry):
      base = base0 + i * (NB * KC)
      gathers = []
      for j in range(NB):
        pltpu.sync_copy(src_ids.at[pl.ds(base + j * KC, KC)], sidx[j])
        gathers.append(pltpu.async_copy(hsrc.at[sidx[j]], rows[j], gsem[j]))
      for j in range(NB):
        pltpu.sync_copy(dst_ids.at[pl.ds(base + j * KC, KC)], didx[j])
      scatters = []
      for j in range(NB):
        gathers[j].wait()
        scatters.append(
            pltpu.async_copy(rows[j], acc.at[didx[j]], ssem[j], add=True))
      for sc in scatters:
        sc.wait()
      return carry

    lax.fori_loop(0, g, body, 0)
    plsc.subcore_barrier()
    for j in range(rows_t // cp):
      r0 = s * rows_t + j * cp
      pltpu.sync_copy(acc.at[pl.ds(r0, cp)], cbuf)
      pltpu.sync_copy(cbuf, out.at[c].at[pl.ds(r0, cp)])

  return agg_kernel


def _prep_body(degp_ref, x_ref, norms_ref, s0_ref):
  n = x_ref.shape[0]
  d = degp_ref[:, 0:2] + degp_ref[:, 2:4]
  norm = jnp.where(d > 0.0, lax.rsqrt(jnp.maximum(d, 1e-30)), 0.0)
  norms_ref[...] = norm
  s0_ref[...] = x_ref[...] * norm[:n, 0:1]


def _layer_body(agg_ref, norms_ref, w_ref, b_ref, g_ref, bt_ref, out_ref,
                *, concat):
  n = norms_ref.shape[0]
  if concat:
    a = jnp.concatenate([agg_ref[0], agg_ref[1]], axis=1)[:n]
  else:
    a = (agg_ref[0] + agg_ref[1])[:n]
  a = a * norms_ref[:, 1:2]
  z = jnp.dot(a, w_ref[...], preferred_element_type=jnp.float32) + b_ref[...]
  mu = jnp.mean(z, axis=0, keepdims=True)
  xc = z - mu
  var = jnp.mean(xc * xc, axis=0, keepdims=True)
  h = g_ref[...] * xc * lax.rsqrt(var + 1e-5) + bt_ref[...]
  h = jnp.maximum(h, 0.0)
  s = h * norms_ref[:, 0:1]
  half = s.shape[1] // 2
  out_ref[0] = s[:, :half]
  out_ref[1] = s[:, half:]


def _final_body(agg_ref, norms_ref, w_ref, b_ref, g_ref, bt_ref,
                wr0_ref, br0_ref, wr1_ref, br1_ref, wr2_ref, br2_ref,
                out_ref):
  n = norms_ref.shape[0]
  a = jnp.concatenate([agg_ref[0], agg_ref[1]], axis=1)[:n]
  a = a * norms_ref[:, 1:2]
  z = jnp.dot(a, w_ref[...], preferred_element_type=jnp.float32) + b_ref[...]
  mu = jnp.mean(z, axis=0, keepdims=True)
  xc = z - mu
  var = jnp.mean(xc * xc, axis=0, keepdims=True)
  h = g_ref[...] * xc * lax.rsqrt(var + 1e-5) + bt_ref[...]
  h = jnp.maximum(h, 0.0)
  hg = jnp.mean(h, axis=0, keepdims=True)
  y = jnp.maximum(jnp.dot(hg, wr0_ref[...],
                          preferred_element_type=jnp.float32) + br0_ref[...], 0.0)
  y = jnp.maximum(jnp.dot(y, wr1_ref[...],
                          preferred_element_type=jnp.float32) + br1_ref[...], 0.0)
  y = jnp.dot(y, wr2_ref[...], preferred_element_type=jnp.float32) + br2_ref[...]
  m = jnp.max(y, axis=0, keepdims=True)
  lse = m + jnp.log(jnp.sum(jnp.exp(y - m), axis=0, keepdims=True))
  out_ref[...] = y - lse


def kernel(x, edge_index, W0, b0, W1, b1, W2, b2, W3, b3, gamma, beta,
           Wr0, br0, Wr1, br1, Wr2, br2):
  n, in_feats = x.shape
  e = edge_index.shape[1]
  h_dim = W0.shape[1]
  npad = 10240

  src_ids = edge_index[0]
  dst_ids = edge_index[1]
  degp = _make_deg_kernel(e)(src_ids, dst_ids)
  degp_col = jnp.transpose(degp.reshape(4, npad))
  norms_pad, s0 = pl.pallas_call(
      _prep_body,
      out_shape=[
          jax.ShapeDtypeStruct((npad, 2), jnp.float32),
          jax.ShapeDtypeStruct((n, in_feats), jnp.float32),
      ],
  )(degp_col, x)
  norms = norms_pad[:n]

  epad = 327680
  assert epad % (NC * NS * 4 * 64) == 0 and epad >= e
  p = epad - e
  pad_src = (jnp.arange(p, dtype=jnp.int32) * 131) % n
  pad_dst = n + (jnp.arange(p, dtype=jnp.int32) % (npad - n))
  src_p = jnp.concatenate([src_ids, pad_src])
  dst_p = jnp.concatenate([dst_ids, pad_dst])
  agg_in = _make_agg_kernel(n, epad, in_feats, edge_split=True)
  agg_h = _make_agg_kernel(n, epad, h_dim // 2)
  out_split = jax.ShapeDtypeStruct((NC, n, h_dim // 2), jnp.float32)
  layer0_call = pl.pallas_call(
      functools.partial(_layer_body, concat=False), out_shape=out_split)
  layer_call = pl.pallas_call(
      functools.partial(_layer_body, concat=True), out_shape=out_split)

  a0 = agg_in(s0, src_p, dst_p)
  s1 = layer0_call(a0, norms, W0, b0, gamma, beta)
  a1 = agg_h(s1, src_p, dst_p)
  s2 = layer_call(a1, norms, W1, b1, gamma, beta)
  a2 = agg_h(s2, src_p, dst_p)
  s3 = layer_call(a2, norms, W2, b2, gamma, beta)
  a3 = agg_h(s3, src_p, dst_p)

  out = pl.pallas_call(
      _final_body,
      out_shape=jax.ShapeDtypeStruct((1, Wr2.shape[1]), jnp.float32),
  )(a3, norms, W3, b3, gamma, beta, Wr0, br0, Wr1, br1, Wr2, br2)
  return out


# speedup vs baseline: 1.3051x; 1.0385x over previous
import functools

import jax
import jax.numpy as jnp
from jax import lax
from jax.experimental import pallas as pl
from jax.experimental.pallas import tpu as pltpu
from jax.experimental.pallas import tpu_sc as plsc

NC = 2
NS = 16
L = 16
K = 80

_mesh = functools.partial(
    plsc.VectorSubcoreMesh, core_axis_name="c", subcore_axis_name="s",
    num_cores=NC, num_subcores=NS)


def _fill(ref, n, value):
  v = jnp.full((L,), value, ref.dtype)
  for k in range(n // L):
    ref[pl.ds(k * L, L)] = v


def _make_deg_kernel(e):
  npad = 10240
  KC = 128
  NB = 2
  per_w = e // (NC * NS)
  g = per_w // (NB * KC)
  assert per_w % (NB * KC) == 0

  @functools.partial(
      pl.kernel,
      out_type=jax.ShapeDtypeStruct((NC * 2 * npad,), jnp.float32),
      mesh=_mesh(),
      scratch_types=[
          [pltpu.VMEM((KC,), jnp.int32) for _ in range(NB)],
          [pltpu.VMEM((KC,), jnp.int32) for _ in range(NB)],
          pltpu.VMEM((KC,), jnp.float32),
          pltpu.VMEM((640,), jnp.float32),
          pltpu.VMEM_SHARED((npad,), jnp.float32),
          pltpu.VMEM_SHARED((npad,), jnp.float32),
          [pltpu.SemaphoreType.DMA for _ in range(2 * NB)],
      ],
  )
  def deg_kernel(src_ids, dst_ids, out, sidx, didx, ones, cbuf, h_out, h_in,
                 sems):
    c = lax.axis_index("c")
    s = lax.axis_index("s")
    _fill(ones, KC, 1.0)
    _fill(cbuf, 640, 0.0)
    pltpu.sync_copy(cbuf, h_out.at[pl.ds(s * 640, 640)])
    pltpu.sync_copy(cbuf, h_in.at[pl.ds(s * 640, 640)])
    plsc.subcore_barrier()

    base0 = (c * NS + s) * per_w

    def body(i, carry):
      base = base0 + i * (NB * KC)
      for j in range(NB):
        pltpu.sync_copy(src_ids.at[pl.ds(base + j * KC, KC)], sidx[j])
        pltpu.sync_copy(dst_ids.at[pl.ds(base + j * KC, KC)], didx[j])
      adds = []
      for j in range(NB):
        adds.append(
            pltpu.async_copy(ones, h_out.at[sidx[j]], sems[2 * j], add=True))
        adds.append(
            pltpu.async_copy(ones, h_in.at[didx[j]], sems[2 * j + 1],
                             add=True))
      for a in adds:
        a.wait()
      return carry

    lax.fori_loop(0, g, body, 0)
    plsc.subcore_barrier()
    pltpu.sync_copy(h_out.at[pl.ds(s * 640, 640)],
                    out.at[pl.ds(c * 2 * npad + s * 640, 640)])
    pltpu.sync_copy(h_in.at[pl.ds(s * 640, 640)],
                    out.at[pl.ds(c * 2 * npad + npad + s * 640, 640)])

  return deg_kernel


def _make_agg_kernel(n, e, hc, edge_split=False):
  KC = 128
  NB = 2
  per_t = e // (NC * NS) if edge_split else e // NS
  g = per_t // (NB * KC)
  assert per_t % (NB * KC) == 0
  npad = 10240
  assert npad >= n
  rows_t = npad // NS
  cp = 64
  assert rows_t % cp == 0

  @functools.partial(
      pl.kernel,
      out_type=jax.ShapeDtypeStruct((NC, npad, hc), jnp.float32),
      mesh=_mesh(),
      scratch_types=[
          [pltpu.VMEM((KC,), jnp.int32) for _ in range(NB)],
          [pltpu.VMEM((KC,), jnp.int32) for _ in range(NB)],
          [pltpu.VMEM((KC, hc), jnp.float32) for _ in range(NB)],
          pltpu.VMEM((cp, hc), jnp.float32),
          pltpu.VMEM_SHARED((npad, hc), jnp.float32),
          [pltpu.SemaphoreType.DMA for _ in range(NB)],
          [pltpu.SemaphoreType.DMA for _ in range(NB)],
      ],
  )
  def agg_kernel(h, src_ids, dst_ids, out, sidx, didx, rows, cbuf, acc,
                 gsem, ssem):
    c = lax.axis_index("c")
    s = lax.axis_index("s")
    hsrc = h if edge_split else h.at[c]

    def zbody(r, carry):
      z = jnp.zeros((L,), jnp.float32)
      for k in range(hc // L):
        cbuf[r, pl.ds(k * L, L)] = z
      return carry
    lax.fori_loop(0, cp, zbody, 0)
    for j in range(rows_t // cp):
      pltpu.sync_copy(cbuf, acc.at[pl.ds(s * rows_t + j * cp, cp)])
    plsc.subcore_barrier()

    base0 = ((c * NS + s) if edge_split else s) * per_t

    def body(i, carry):
      base = base0 + i * (NB * KC)
      gathers = []
      for j in range(NB):
        pltpu.sync_copy(src_ids.at[pl.ds(base + j * KC, KC)], sidx[j])
        gathers.append(pltpu.async_copy(hsrc.at[sidx[j]], rows[j], gsem[j]))
      for j in range(NB):
        pltpu.sync_copy(dst_ids.at[pl.ds(base + j * KC, KC)], didx[j])
      scatters = []
      for j in range(NB):
        gathers[j].wait()
        scatters.append(
            pltpu.async_copy(rows[j], acc.at[didx[j]], ssem[j], add=True))
      for sc in scatters:
        sc.wait()
      return carry

    lax.fori_loop(0, g, body, 0)
    plsc.subcore_barrier()
    pltpu.sync_copy(acc.at[pl.ds(s * rows_t, rows_t)],
                    out.at[c].at[pl.ds(s * rows_t, rows_t)])

  return agg_kernel


def _prep_body(degp_ref, x_ref, norms_ref, s0_ref):
  n = x_ref.shape[0]
  d = degp_ref[:, 0:2] + degp_ref[:, 2:4]
  norm = jnp.where(d > 0.0, lax.rsqrt(jnp.maximum(d, 1e-30)), 0.0)
  norms_ref[...] = norm
  s0_ref[...] = x_ref[...] * norm[:n, 0:1]


def _layer_body(agg_ref, norms_ref, w_ref, b_ref, g_ref, bt_ref, out_ref,
                *, concat):
  n = norms_ref.shape[0]
  if concat:
    a = jnp.concatenate([agg_ref[0], agg_ref[1]], axis=1)[:n]
  else:
    a = (agg_ref[0] + agg_ref[1])[:n]
  a = a * norms_ref[:, 1:2]
  z = jnp.dot(a, w_ref[...], preferred_element_type=jnp.float32) + b_ref[...]
  mu = jnp.mean(z, axis=0, keepdims=True)
  xc = z - mu
  var = jnp.mean(xc * xc, axis=0, keepdims=True)
  h = g_ref[...] * xc * lax.rsqrt(var + 1e-5) + bt_ref[...]
  h = jnp.maximum(h, 0.0)
  s = h * norms_ref[:, 0:1]
  half = s.shape[1] // 2
  out_ref[0] = s[:, :half]
  out_ref[1] = s[:, half:]


def _final_body(agg_ref, norms_ref, w_ref, b_ref, g_ref, bt_ref,
                wr0_ref, br0_ref, wr1_ref, br1_ref, wr2_ref, br2_ref,
                out_ref):
  n = norms_ref.shape[0]
  a = jnp.concatenate([agg_ref[0], agg_ref[1]], axis=1)[:n]
  a = a * norms_ref[:, 1:2]
  z = jnp.dot(a, w_ref[...], preferred_element_type=jnp.float32) + b_ref[...]
  mu = jnp.mean(z, axis=0, keepdims=True)
  xc = z - mu
  var = jnp.mean(xc * xc, axis=0, keepdims=True)
  h = g_ref[...] * xc * lax.rsqrt(var + 1e-5) + bt_ref[...]
  h = jnp.maximum(h, 0.0)
  hg = jnp.mean(h, axis=0, keepdims=True)
  y = jnp.maximum(jnp.dot(hg, wr0_ref[...],
                          preferred_element_type=jnp.float32) + br0_ref[...], 0.0)
  y = jnp.maximum(jnp.dot(y, wr1_ref[...],
                          preferred_element_type=jnp.float32) + br1_ref[...], 0.0)
  y = jnp.dot(y, wr2_ref[...], preferred_element_type=jnp.float32) + br2_ref[...]
  m = jnp.max(y, axis=0, keepdims=True)
  lse = m + jnp.log(jnp.sum(jnp.exp(y - m), axis=0, keepdims=True))
  out_ref[...] = y - lse


def kernel(x, edge_index, W0, b0, W1, b1, W2, b2, W3, b3, gamma, beta,
           Wr0, br0, Wr1, br1, Wr2, br2):
  n, in_feats = x.shape
  e = edge_index.shape[1]
  h_dim = W0.shape[1]
  npad = 10240

  src_ids = edge_index[0]
  dst_ids = edge_index[1]
  epad = 327680
  assert epad % (NC * NS * 4 * 64) == 0 and epad >= e
  p = epad - e
  pad_dst = n + (jnp.arange(p, dtype=jnp.int32) % (npad - n))
  src_d = jnp.concatenate([src_ids, pad_dst])
  dst_d = jnp.concatenate([dst_ids, pad_dst])
  degp = _make_deg_kernel(epad)(src_d, dst_d)
  degp_col = jnp.transpose(degp.reshape(4, npad))
  norms_pad, s0 = pl.pallas_call(
      _prep_body,
      out_shape=[
          jax.ShapeDtypeStruct((npad, 2), jnp.float32),
          jax.ShapeDtypeStruct((n, in_feats), jnp.float32),
      ],
  )(degp_col, x)
  norms = norms_pad[:n]

  pad_src = (jnp.arange(p, dtype=jnp.int32) * 131) % n
  src_p = jnp.concatenate([src_ids, pad_src])
  dst_p = dst_d
  agg_in = _make_agg_kernel(n, epad, in_feats, edge_split=True)
  agg_h = _make_agg_kernel(n, epad, h_dim // 2)
  out_split = jax.ShapeDtypeStruct((NC, n, h_dim // 2), jnp.float32)
  layer0_call = pl.pallas_call(
      functools.partial(_layer_body, concat=False), out_shape=out_split)
  layer_call = pl.pallas_call(
      functools.partial(_layer_body, concat=True), out_shape=out_split)

  a0 = agg_in(s0, src_p, dst_p)
  s1 = layer0_call(a0, norms, W0, b0, gamma, beta)
  a1 = agg_h(s1, src_p, dst_p)
  s2 = layer_call(a1, norms, W1, b1, gamma, beta)
  a2 = agg_h(s2, src_p, dst_p)
  s3 = layer_call(a2, norms, W2, b2, gamma, beta)
  a3 = agg_h(s3, src_p, dst_p)

  out = pl.pallas_call(
      _final_body,
      out_shape=jax.ShapeDtypeStruct((1, Wr2.shape[1]), jnp.float32),
  )(a3, norms, W3, b3, gamma, beta, Wr0, br0, Wr1, br1, Wr2, br2)
  return out
